# baseline (speedup 1.0000x reference)
import functools

import jax
import jax.numpy as jnp
from jax import lax
from jax.experimental import pallas as pl
from jax.experimental.pallas import tpu as pltpu
from jax.experimental.pallas import tpu_sc as plsc

N = 50000
E = 800000
D = 64
HD = 32
G = 128

NC, NS = 2, 16
EK = 80
SUB = 5
CHUNK = EK * SUB
EPT = E // NS
OUTER = EPT // CHUNK
ROWS_PT = N // NS

BK = 1000
NBLK = N // BK

_f32 = jnp.float32


def _run_half(s, x_hbm, out_hbm, src2, dst2,
              acc, sidx0, didx0, rows0, sidx1, didx1, rows1, gsem0, gsem1):
    pltpu.sync_copy(x_hbm.at[pl.ds(s * ROWS_PT, ROWS_PT)],
                    acc.at[pl.ds(s * ROWS_PT, ROWS_PT)])
    plsc.subcore_barrier()

    def fire(i, sx, dx, rw, sem):
        row0 = s * (EPT // EK) + i * SUB
        pltpu.sync_copy(src2.at[pl.ds(row0, SUB)], sx)
        pltpu.sync_copy(dst2.at[pl.ds(row0, SUB)], dx)
        for j in range(SUB):
            pltpu.async_copy(x_hbm.at[sx.at[j]], rw.at[j], sem)

    def drain_scatter(sx, dx, rw, sem):
        for j in range(SUB):
            pltpu.make_async_copy(x_hbm.at[sx.at[j]], rw.at[j], sem).wait()
        for j in range(SUB):
            pltpu.sync_copy(rw.at[j], acc.at[dx.at[j]], add=True)

    fire(0, sidx0, didx0, rows0, gsem0)

    def outer(k, carry):
        i = 2 * k
        fire(i + 1, sidx1, didx1, rows1, gsem1)
        drain_scatter(sidx0, didx0, rows0, gsem0)
        fire(i + 2, sidx0, didx0, rows0, gsem0)
        drain_scatter(sidx1, didx1, rows1, gsem1)
        return carry
    lax.fori_loop(0, (OUTER - 1) // 2, outer, 0)
    drain_scatter(sidx0, didx0, rows0, gsem0)

    plsc.subcore_barrier()
    pltpu.sync_copy(acc.at[pl.ds(s * ROWS_PT, ROWS_PT)],
                    out_hbm.at[pl.ds(s * ROWS_PT, ROWS_PT)])


def _sc_agg_body(xlo, xhi, src2, dst2, mlo, mhi,
                 acc, sidx0, didx0, rows0, sidx1, didx1, rows1, gsem0, gsem1):
    c = lax.axis_index("c")
    s = lax.axis_index("s")
    bufs = (acc, sidx0, didx0, rows0, sidx1, didx1, rows1, gsem0, gsem1)

    @pl.when(c == 0)
    def _():
        _run_half(s, xlo, mlo, src2, dst2, *bufs)

    @pl.when(c == 1)
    def _():
        _run_half(s, xhi, mhi, src2, dst2, *bufs)


def _sc_agg1_body(x, src2, dst2, m,
                  acc, sidx0, didx0, rows0, sidx1, didx1, rows1, gsem0, gsem1):
    s = lax.axis_index("s")
    _run_half(s, x, m, src2, dst2,
              acc, sidx0, didx0, rows0, sidx1, didx1, rows1, gsem0, gsem1)


_SC_SCRATCH = [
    pltpu.VMEM_SHARED((N, HD), _f32),
    pltpu.VMEM((SUB, EK), jnp.int32),
    pltpu.VMEM((SUB, EK), jnp.int32),
    pltpu.VMEM((SUB, EK, HD), _f32),
    pltpu.VMEM((SUB, EK), jnp.int32),
    pltpu.VMEM((SUB, EK), jnp.int32),
    pltpu.VMEM((SUB, EK, HD), _f32),
    pltpu.SemaphoreType.DMA,
    pltpu.SemaphoreType.DMA,
]

_sc_agg = pl.kernel(
    _sc_agg_body,
    out_type=(jax.ShapeDtypeStruct((N, HD), _f32),
              jax.ShapeDtypeStruct((N, HD), _f32)),
    mesh=plsc.VectorSubcoreMesh(core_axis_name="c", subcore_axis_name="s",
                                num_cores=NC, num_subcores=NS),
    scratch_types=list(_SC_SCRATCH),
    compiler_params=pltpu.CompilerParams(use_tc_tiling_on_sc=False),
)

_sc_agg1 = pl.kernel(
    _sc_agg1_body,
    out_type=jax.ShapeDtypeStruct((N, HD), _f32),
    mesh=plsc.VectorSubcoreMesh(core_axis_name="c", subcore_axis_name="s",
                                num_cores=1, num_subcores=NS),
    scratch_types=list(_SC_SCRATCH),
    compiler_params=pltpu.CompilerParams(use_tc_tiling_on_sc=False),
)


def _pass1_body(mlo, mhi, w1, b1, y, ssum, ssq):
    i = pl.program_id(0)
    h = jnp.concatenate([mlo[...], mhi[...]], axis=1)
    yv = jnp.dot(h, w1[...], preferred_element_type=_f32) + b1[...]
    y[...] = yv

    @pl.when(i == 0)
    def _():
        ssum[...] = jnp.zeros_like(ssum)
        ssq[...] = jnp.zeros_like(ssq)

    ssum[...] += jnp.sum(yv, axis=0, keepdims=True)
    ssq[...] += jnp.sum(yv * yv, axis=0, keepdims=True)


_tc_pass1 = pl.pallas_call(
    _pass1_body,
    grid=(NBLK,),
    in_specs=[
        pl.BlockSpec((BK, HD), lambda i: (i, 0)),
        pl.BlockSpec((BK, HD), lambda i: (i, 0)),
        pl.BlockSpec((D, D), lambda i: (0, 0)),
        pl.BlockSpec((1, D), lambda i: (0, 0)),
    ],
    out_specs=[
        pl.BlockSpec((BK, D), lambda i: (i, 0)),
        pl.BlockSpec((1, D), lambda i: (0, 0)),
        pl.BlockSpec((1, D), lambda i: (0, 0)),
    ],
    out_shape=[
        jax.ShapeDtypeStruct((N, D), _f32),
        jax.ShapeDtypeStruct((1, D), _f32),
        jax.ShapeDtypeStruct((1, D), _f32),
    ],
)


def _bn_mlp(y, ssum, ssq, gamma, beta, w2, b2):
    mean = ssum[...] * (1.0 / N)
    var = ssq[...] * (1.0 / N) - mean * mean
    inv = lax.rsqrt(var + 1e-5) * gamma[...]
    z = jnp.maximum((y[...] - mean) * inv + beta[...], 0.0)
    hv = jnp.dot(z, w2[...], preferred_element_type=_f32) + b2[...]
    return jnp.maximum(hv, 0.0)


def _pass2_body(y, ssum, ssq, gamma, beta, w2, b2, hlo, hhi):
    hv = _bn_mlp(y, ssum, ssq, gamma, beta, w2, b2)
    hlo[...] = hv[:, :HD]
    hhi[...] = hv[:, HD:]


_tc_pass2 = pl.pallas_call(
    _pass2_body,
    grid=(NBLK,),
    in_specs=[
        pl.BlockSpec((BK, D), lambda i: (i, 0)),
        pl.BlockSpec((1, D), lambda i: (0, 0)),
        pl.BlockSpec((1, D), lambda i: (0, 0)),
        pl.BlockSpec((1, D), lambda i: (0, 0)),
        pl.BlockSpec((1, D), lambda i: (0, 0)),
        pl.BlockSpec((D, D), lambda i: (0, 0)),
        pl.BlockSpec((1, D), lambda i: (0, 0)),
    ],
    out_specs=[
        pl.BlockSpec((BK, HD), lambda i: (i, 0)),
        pl.BlockSpec((BK, HD), lambda i: (i, 0)),
    ],
    out_shape=[
        jax.ShapeDtypeStruct((N, HD), _f32),
        jax.ShapeDtypeStruct((N, HD), _f32),
    ],
)


def _pass2_pool_body(y, ssum, ssq, gamma, beta, w2, b2,
                     batch3, batchc, wc1, bc1, wc2, bc2, out,
                     psum, pcnt, pmax):
    i = pl.program_id(0)

    @pl.when(i == 0)
    def _():
        psum[...] = jnp.zeros_like(psum)
        pcnt[...] = jnp.zeros_like(pcnt)
        pmax[...] = jnp.full_like(pmax, -jnp.inf)

    h = _bn_mlp(y, ssum, ssq, gamma, beta, w2, b2)

    brow = batch3[0]
    bcol = batchc[...]
    gids = lax.broadcasted_iota(jnp.int32, (G, BK), 0)
    oh = (gids == brow).astype(_f32)
    psum[...] += jnp.dot(oh, h, preferred_element_type=_f32)
    pcnt[...] += jnp.dot(oh, jnp.ones((BK, 8), _f32),
                         preferred_element_type=_f32)

    gmin = batch3[0, 0, 0]
    gmax = batch3[0, 0, BK - 1]

    def mb(g, carry):
        @pl.when((g >= gmin) & (g <= gmax))
        def _():
            m = jnp.where(bcol == g, h, -jnp.inf)
            cm = jnp.max(m, axis=0, keepdims=True)
            pmax[pl.ds(g, 1), :] = jnp.maximum(pmax[pl.ds(g, 1), :], cm)
        return carry
    lax.fori_loop(0, G, mb, 0)

    @pl.when(i == NBLK - 1)
    def _():
        mean_pool = psum[...] / jnp.maximum(pcnt[:, 0:1], 1.0)
        gg = jnp.concatenate([mean_pool, pmax[...]], axis=1)
        t = jnp.maximum(
            jnp.dot(gg, wc1[...], preferred_element_type=_f32) + bc1[...], 0.0)
        out[...] = jnp.dot(t, wc2[...], preferred_element_type=_f32) + bc2[...]


_tc_pass2_pool = pl.pallas_call(
    _pass2_pool_body,
    grid=(NBLK,),
    in_specs=[
        pl.BlockSpec((BK, D), lambda i: (i, 0)),
        pl.BlockSpec((1, D), lambda i: (0, 0)),
        pl.BlockSpec((1, D), lambda i: (0, 0)),
        pl.BlockSpec((1, D), lambda i: (0, 0)),
        pl.BlockSpec((1, D), lambda i: (0, 0)),
        pl.BlockSpec((D, D), lambda i: (0, 0)),
        pl.BlockSpec((1, D), lambda i: (0, 0)),
        pl.BlockSpec((1, 1, BK), lambda i: (i, 0, 0)),
        pl.BlockSpec((BK, 1), lambda i: (i, 0)),
        pl.BlockSpec((2 * D, D), lambda i: (0, 0)),
        pl.BlockSpec((1, D), lambda i: (0, 0)),
        pl.BlockSpec((D, 128), lambda i: (0, 0)),
        pl.BlockSpec((1, 128), lambda i: (0, 0)),
    ],
    out_specs=pl.BlockSpec((G, 128), lambda i: (0, 0)),
    out_shape=jax.ShapeDtypeStruct((G, 128), _f32),
    scratch_shapes=[
        pltpu.VMEM((G, D), _f32),
        pltpu.VMEM((G, 8), _f32),
        pltpu.VMEM((G, D), _f32),
    ],
)


def kernel(x, edge_index, batch, params):
    src2 = edge_index[0].reshape(E // EK, EK)
    dst2 = edge_index[1].reshape(E // EK, EK)
    h_lo = x[:, :HD]
    h_hi = x[:, HD:]
    c = params["cls"]
    batch3 = batch.reshape(NBLK, 1, BK)
    batchc = batch.reshape(N, 1)
    w2p = jnp.pad(c["W2"], ((0, 0), (0, 128 - 2)))
    b2p = jnp.pad(c["b2"].reshape(1, 2), ((0, 0), (0, 128 - 2)))

    for li, name in enumerate(("conv1", "conv2", "conv3")):
        p = params[name]
        if li == 0:
            m_lo, m_hi = _sc_agg(h_lo, h_hi, src2, dst2)
        else:
            m_lo = _sc_agg1(h_lo, src2, dst2)
            m_hi = _sc_agg1(h_hi, src2, dst2)
        y, ssum, ssq = _tc_pass1(m_lo, m_hi, p["W1"], p["b1"].reshape(1, D))
        bn_args = (y, ssum, ssq, p["gamma"].reshape(1, D),
                   p["beta"].reshape(1, D), p["W2"], p["b2"].reshape(1, D))
        if li < 2:
            h_lo, h_hi = _tc_pass2(*bn_args)
        else:
            outp = _tc_pass2_pool(*bn_args, batch3, batchc, c["W1"],
                                  c["b1"].reshape(1, D), w2p, b2p)
    return outp[:, :2]


# speedup vs baseline: 1.3912x; 1.3912x over previous
import functools

import jax
import jax.numpy as jnp
from jax import lax
from jax.experimental import pallas as pl
from jax.experimental.pallas import tpu as pltpu
from jax.experimental.pallas import tpu_sc as plsc

N = 50000
E = 800000
D = 64
HD = 32
G = 128

NC, NS = 2, 16
EK = 80
SUB = 5
CHUNK = EK * SUB
EPT = E // NS
OUTER = EPT // CHUNK
ROWS_PT = N // NS

BK = 1000
NBLK = N // BK

_f32 = jnp.float32


def _run_half(s, x_hbm, out_hbm, src2, dst2,
              acc, sidx0, didx0, rows0, sidx1, didx1, rows1,
              gsem0, gsem1, ssem0, ssem1):
    pltpu.sync_copy(x_hbm.at[pl.ds(s * ROWS_PT, ROWS_PT)],
                    acc.at[pl.ds(s * ROWS_PT, ROWS_PT)])
    plsc.subcore_barrier()

    NIR = EPT // EK

    def _row0(i):
        return jnp.minimum(s * NIR + i * SUB, (E // EK) - SUB)

    def stage_fire(i, sx, dx, ssem):
        r = _row0(i)
        pltpu.async_copy(src2.at[pl.ds(r, SUB)], sx, ssem)
        pltpu.async_copy(dst2.at[pl.ds(r, SUB)], dx, ssem)

    def stage_wait(i, sx, dx, ssem):
        r = _row0(i)
        pltpu.make_async_copy(src2.at[pl.ds(r, SUB)], sx, ssem).wait()
        pltpu.make_async_copy(dst2.at[pl.ds(r, SUB)], dx, ssem).wait()

    def gather_fire(sx, rw, gsem):
        for j in range(SUB):
            pltpu.async_copy(x_hbm.at[sx.at[j]], rw.at[j], gsem)

    def gather_wait(sx, rw, gsem):
        for j in range(SUB):
            pltpu.make_async_copy(x_hbm.at[sx.at[j]], rw.at[j], gsem).wait()

    def scatter(dx, rw):
        for j in range(SUB):
            pltpu.sync_copy(rw.at[j], acc.at[dx.at[j]], add=True)

    stage_fire(0, sidx0, didx0, ssem0)
    stage_wait(0, sidx0, didx0, ssem0)
    gather_fire(sidx0, rows0, gsem0)
    stage_fire(1, sidx1, didx1, ssem1)

    def outer(k, carry):
        i = 2 * k
        gather_wait(sidx0, rows0, gsem0)
        stage_wait(i + 1, sidx1, didx1, ssem1)
        gather_fire(sidx1, rows1, gsem1)
        scatter(didx0, rows0)
        stage_fire(i + 2, sidx0, didx0, ssem0)
        gather_wait(sidx1, rows1, gsem1)
        scatter(didx1, rows1)
        stage_fire(i + 3, sidx1, didx1, ssem1)
        stage_wait(i + 2, sidx0, didx0, ssem0)
        gather_fire(sidx0, rows0, gsem0)
        return carry
    lax.fori_loop(0, (OUTER - 1) // 2, outer, 0)

    gather_wait(sidx0, rows0, gsem0)
    scatter(didx0, rows0)
    stage_wait(OUTER, sidx1, didx1, ssem1)

    plsc.subcore_barrier()
    pltpu.sync_copy(acc.at[pl.ds(s * ROWS_PT, ROWS_PT)],
                    out_hbm.at[pl.ds(s * ROWS_PT, ROWS_PT)])


def _sc_agg_body(xlo, xhi, src2, dst2, mlo, mhi,
                 acc, sidx0, didx0, rows0, sidx1, didx1, rows1,
                 gsem0, gsem1, ssem0, ssem1):
    c = lax.axis_index("c")
    s = lax.axis_index("s")
    bufs = (acc, sidx0, didx0, rows0, sidx1, didx1, rows1,
            gsem0, gsem1, ssem0, ssem1)

    @pl.when(c == 0)
    def _():
        _run_half(s, xlo, mlo, src2, dst2, *bufs)

    @pl.when(c == 1)
    def _():
        _run_half(s, xhi, mhi, src2, dst2, *bufs)


_SC_SCRATCH = [
    pltpu.VMEM_SHARED((N, HD), _f32),
    pltpu.VMEM((SUB, EK), jnp.int32),
    pltpu.VMEM((SUB, EK), jnp.int32),
    pltpu.VMEM((SUB, EK, HD), _f32),
    pltpu.VMEM((SUB, EK), jnp.int32),
    pltpu.VMEM((SUB, EK), jnp.int32),
    pltpu.VMEM((SUB, EK, HD), _f32),
    pltpu.SemaphoreType.DMA,
    pltpu.SemaphoreType.DMA,
    pltpu.SemaphoreType.DMA,
    pltpu.SemaphoreType.DMA,
]

_sc_agg = pl.kernel(
    _sc_agg_body,
    out_type=(jax.ShapeDtypeStruct((N, HD), _f32),
              jax.ShapeDtypeStruct((N, HD), _f32)),
    mesh=plsc.VectorSubcoreMesh(core_axis_name="c", subcore_axis_name="s",
                                num_cores=NC, num_subcores=NS),
    scratch_types=list(_SC_SCRATCH),
    compiler_params=pltpu.CompilerParams(use_tc_tiling_on_sc=False),
)

def _pass1_body(mlo, mhi, w1, b1, y, ssum, ssq):
    i = pl.program_id(0)
    h = jnp.concatenate([mlo[...], mhi[...]], axis=1)
    yv = jnp.dot(h, w1[...], preferred_element_type=_f32) + b1[...]
    y[...] = yv

    @pl.when(i == 0)
    def _():
        ssum[...] = jnp.zeros_like(ssum)
        ssq[...] = jnp.zeros_like(ssq)

    ssum[...] += jnp.sum(yv, axis=0, keepdims=True)
    ssq[...] += jnp.sum(yv * yv, axis=0, keepdims=True)


_tc_pass1 = pl.pallas_call(
    _pass1_body,
    grid=(NBLK,),
    in_specs=[
        pl.BlockSpec((BK, HD), lambda i: (i, 0)),
        pl.BlockSpec((BK, HD), lambda i: (i, 0)),
        pl.BlockSpec((D, D), lambda i: (0, 0)),
        pl.BlockSpec((1, D), lambda i: (0, 0)),
    ],
    out_specs=[
        pl.BlockSpec((BK, D), lambda i: (i, 0)),
        pl.BlockSpec((1, D), lambda i: (0, 0)),
        pl.BlockSpec((1, D), lambda i: (0, 0)),
    ],
    out_shape=[
        jax.ShapeDtypeStruct((N, D), _f32),
        jax.ShapeDtypeStruct((1, D), _f32),
        jax.ShapeDtypeStruct((1, D), _f32),
    ],
)


def _bn_mlp(y, ssum, ssq, gamma, beta, w2, b2):
    mean = ssum[...] * (1.0 / N)
    var = ssq[...] * (1.0 / N) - mean * mean
    inv = lax.rsqrt(var + 1e-5) * gamma[...]
    z = jnp.maximum((y[...] - mean) * inv + beta[...], 0.0)
    hv = jnp.dot(z, w2[...], preferred_element_type=_f32) + b2[...]
    return jnp.maximum(hv, 0.0)


def _pass2_body(y, ssum, ssq, gamma, beta, w2, b2, hlo, hhi):
    hv = _bn_mlp(y, ssum, ssq, gamma, beta, w2, b2)
    hlo[...] = hv[:, :HD]
    hhi[...] = hv[:, HD:]


_tc_pass2 = pl.pallas_call(
    _pass2_body,
    grid=(NBLK,),
    in_specs=[
        pl.BlockSpec((BK, D), lambda i: (i, 0)),
        pl.BlockSpec((1, D), lambda i: (0, 0)),
        pl.BlockSpec((1, D), lambda i: (0, 0)),
        pl.BlockSpec((1, D), lambda i: (0, 0)),
        pl.BlockSpec((1, D), lambda i: (0, 0)),
        pl.BlockSpec((D, D), lambda i: (0, 0)),
        pl.BlockSpec((1, D), lambda i: (0, 0)),
    ],
    out_specs=[
        pl.BlockSpec((BK, HD), lambda i: (i, 0)),
        pl.BlockSpec((BK, HD), lambda i: (i, 0)),
    ],
    out_shape=[
        jax.ShapeDtypeStruct((N, HD), _f32),
        jax.ShapeDtypeStruct((N, HD), _f32),
    ],
)


def _pass2_pool_body(y, ssum, ssq, gamma, beta, w2, b2,
                     batch3, batchc, wc1, bc1, wc2, bc2, out,
                     psum, pcnt, pmax):
    i = pl.program_id(0)

    @pl.when(i == 0)
    def _():
        psum[...] = jnp.zeros_like(psum)
        pcnt[...] = jnp.zeros_like(pcnt)
        pmax[...] = jnp.full_like(pmax, -jnp.inf)

    h = _bn_mlp(y, ssum, ssq, gamma, beta, w2, b2)

    brow = batch3[0]
    bcol = batchc[...]
    gids = lax.broadcasted_iota(jnp.int32, (G, BK), 0)
    oh = (gids == brow).astype(_f32)
    psum[...] += jnp.dot(oh, h, preferred_element_type=_f32)
    pcnt[...] += jnp.dot(oh, jnp.ones((BK, 8), _f32),
                         preferred_element_type=_f32)

    gmin = batch3[0, 0, 0]
    gmax = batch3[0, 0, BK - 1]

    def mb(g, carry):
        @pl.when((g >= gmin) & (g <= gmax))
        def _():
            m = jnp.where(bcol == g, h, -jnp.inf)
            cm = jnp.max(m, axis=0, keepdims=True)
            pmax[pl.ds(g, 1), :] = jnp.maximum(pmax[pl.ds(g, 1), :], cm)
        return carry
    lax.fori_loop(0, G, mb, 0)

    @pl.when(i == NBLK - 1)
    def _():
        mean_pool = psum[...] / jnp.maximum(pcnt[:, 0:1], 1.0)
        gg = jnp.concatenate([mean_pool, pmax[...]], axis=1)
        t = jnp.maximum(
            jnp.dot(gg, wc1[...], preferred_element_type=_f32) + bc1[...], 0.0)
        out[...] = jnp.dot(t, wc2[...], preferred_element_type=_f32) + bc2[...]


_tc_pass2_pool = pl.pallas_call(
    _pass2_pool_body,
    grid=(NBLK,),
    in_specs=[
        pl.BlockSpec((BK, D), lambda i: (i, 0)),
        pl.BlockSpec((1, D), lambda i: (0, 0)),
        pl.BlockSpec((1, D), lambda i: (0, 0)),
        pl.BlockSpec((1, D), lambda i: (0, 0)),
        pl.BlockSpec((1, D), lambda i: (0, 0)),
        pl.BlockSpec((D, D), lambda i: (0, 0)),
        pl.BlockSpec((1, D), lambda i: (0, 0)),
        pl.BlockSpec((1, 1, BK), lambda i: (i, 0, 0)),
        pl.BlockSpec((BK, 1), lambda i: (i, 0)),
        pl.BlockSpec((2 * D, D), lambda i: (0, 0)),
        pl.BlockSpec((1, D), lambda i: (0, 0)),
        pl.BlockSpec((D, 128), lambda i: (0, 0)),
        pl.BlockSpec((1, 128), lambda i: (0, 0)),
    ],
    out_specs=pl.BlockSpec((G, 128), lambda i: (0, 0)),
    out_shape=jax.ShapeDtypeStruct((G, 128), _f32),
    scratch_shapes=[
        pltpu.VMEM((G, D), _f32),
        pltpu.VMEM((G, 8), _f32),
        pltpu.VMEM((G, D), _f32),
    ],
)


def kernel(x, edge_index, batch, params):
    src2 = edge_index[0].reshape(E // EK, EK)
    dst2 = edge_index[1].reshape(E // EK, EK)
    h_lo = x[:, :HD]
    h_hi = x[:, HD:]
    c = params["cls"]
    batch3 = batch.reshape(NBLK, 1, BK)
    batchc = batch.reshape(N, 1)
    w2p = jnp.pad(c["W2"], ((0, 0), (0, 128 - 2)))
    b2p = jnp.pad(c["b2"].reshape(1, 2), ((0, 0), (0, 128 - 2)))

    for li, name in enumerate(("conv1", "conv2", "conv3")):
        p = params[name]
        m_lo, m_hi = _sc_agg(h_lo, h_hi, src2, dst2)
        y, ssum, ssq = _tc_pass1(m_lo, m_hi, p["W1"], p["b1"].reshape(1, D))
        bn_args = (y, ssum, ssq, p["gamma"].reshape(1, D),
                   p["beta"].reshape(1, D), p["W2"], p["b2"].reshape(1, D))
        if li < 2:
            h_lo, h_hi = _tc_pass2(*bn_args)
        else:
            outp = _tc_pass2_pool(*bn_args, batch3, batchc, c["W1"],
                                  c["b1"].reshape(1, D), w2p, b2p)
    return outp[:, :2]
